# baseline (speedup 1.0000x reference)
import functools

import jax
import jax.numpy as jnp
from jax import lax
from jax.experimental import pallas as pl
from jax.experimental.pallas import tpu as pltpu
from jax.experimental.pallas import tpu_sc as plsc

_NC = 2
_NS = 16
_NW = _NC * _NS
_L = 16


@functools.lru_cache(maxsize=None)
def _make_sc_scores(V, D, B, C, NTOT):
    ni = B // _NW
    no = (B * C) // _NW
    nn = (B * NTOT) // _NW
    NH = D // (2 * _L)
    assert nn == ni * NTOT and no == ni * C
    mesh = plsc.VectorSubcoreMesh(core_axis_name="c", subcore_axis_name="s")

    @functools.partial(
        pl.kernel, mesh=mesh,
        out_type=[
            jax.ShapeDtypeStruct((B * C,), jnp.float32),
            jax.ShapeDtypeStruct((B * NTOT,), jnp.float32),
        ],
        scratch_types=[
            pltpu.VMEM((no,), jnp.int32),
            pltpu.VMEM((nn,), jnp.int32),
            pltpu.VMEM((ni, D), jnp.float32),
            pltpu.VMEM((no, D), jnp.float32),
            pltpu.VMEM((NTOT, D), jnp.float32),
            pltpu.VMEM((NTOT, D), jnp.float32),
            pltpu.VMEM((no + _L,), jnp.float32),
            pltpu.VMEM((nn + _L,), jnp.float32),
            pltpu.SemaphoreType.DMA,
            pltpu.SemaphoreType.DMA,
            pltpu.SemaphoreType.DMA,
            pltpu.SemaphoreType.DMA,
            pltpu.SemaphoreType.DMA,
        ],
        compiler_params=pltpu.CompilerParams(
            use_tc_tiling_on_sc=False, needs_layout_passes=False),
    )
    def sgns_sc(iv_all, emb_o, ow, nw, osc_out, nsc_out,
                owi, nwi, iv_v, ow_v, nv0, nv1, sc_o, sc_n,
                semp, sem0, sem0b, sem1, sem1b):
        HK = NTOT // 2

        def start_nv(b, nv_buf, sems):
            base = b * NTOT
            pltpu.async_copy(
                emb_o.at[nwi.at[pl.ds(base, HK)]],
                nv_buf.at[pl.ds(0, HK)], sems[0])
            pltpu.async_copy(
                emb_o.at[nwi.at[pl.ds(base + HK, HK)]],
                nv_buf.at[pl.ds(HK, HK)], sems[1])

        def wait_nv(nv_buf, sems):
            pltpu.make_async_copy(
                emb_o.at[nwi.at[pl.ds(0, HK)]],
                nv_buf.at[pl.ds(0, HK)], sems[0]).wait()
            pltpu.make_async_copy(
                emb_o.at[nwi.at[pl.ds(0, HK)]],
                nv_buf.at[pl.ds(HK, HK)], sems[1]).wait()

        wid = lax.axis_index("s") * _NC + lax.axis_index("c")
        pltpu.sync_copy(nw.at[pl.ds(wid * nn, nn)], nwi)
        start_nv(0, nv0, (sem0, sem0b))
        pltpu.sync_copy(ow.at[pl.ds(wid * no, no)], owi)
        pltpu.sync_copy(iv_all.at[pl.ds(wid * ni, ni)], iv_v)
        pltpu.async_copy(emb_o.at[owi], ow_v, semp).wait()

        lane = lax.broadcasted_iota(jnp.int32, (_L,), 0)

        def row_quarters(rows_ref, r):
            return [rows_ref[r, pl.ds(q * _L, _L)] for q in range(2 * NH)]

        def dots_group(rows_ref, rbase, count, ivq, sc_ref, sbase):
            score = jnp.zeros((_L,), jnp.float32)
            for u in range(count):
                qs = row_quarters(rows_ref, rbase + u)
                p = qs[0] * ivq[0]
                for q in range(1, 2 * NH):
                    p += qs[q] * ivq[q]
                score = jnp.where(lane == u, jnp.sum(p), score)
            sc_ref[pl.ds(sbase, _L)] = score

        def half(b, nv_cur, sem_cur, nv_nxt, sem_nxt, nxt_b, has_next):
            @pl.when(has_next)
            def _():
                start_nv(nxt_b, nv_nxt, sem_nxt)
            ivq = row_quarters(iv_v, b)
            for g0 in range(0, C, _L):
                dots_group(ow_v, b * C + g0, min(_L, C - g0), ivq,
                           sc_o, b * C + g0)
            wait_nv(nv_cur, sem_cur)

            def gbody(jj, cc):
                dots_group(nv_cur, jj * _L, _L, ivq, sc_n,
                           b * NTOT + jj * _L)
                return cc
            lax.fori_loop(0, NTOT // _L, gbody, 0, unroll=5)

        def pair(bb, c):
            b0 = 2 * bb
            half(b0, nv0, (sem0, sem0b), nv1, (sem1, sem1b), b0 + 1, True)
            half(b0 + 1, nv1, (sem1, sem1b), nv0, (sem0, sem0b), b0 + 2,
                 bb < ni // 2 - 1)
            return c

        lax.fori_loop(0, ni // 2, pair, 0)
        pltpu.sync_copy(sc_o.at[pl.ds(0, no)],
                        osc_out.at[pl.ds(wid * no, no)])
        pltpu.sync_copy(sc_n.at[pl.ds(0, nn)],
                        nsc_out.at[pl.ds(wid * nn, nn)])

    return sgns_sc


def _log_sigmoid(x):
    return jnp.minimum(x, 0.0) - jnp.log1p(jnp.exp(-jnp.abs(x)))


@functools.lru_cache(maxsize=None)
def _make_loss(B, C, NTOT):
    scale = -1.0 / (B * C)
    ro = (B * C) // 128
    rn = (B * NTOT) // 128

    def body(osc_ref, nsc_ref, out_ref):
        part = (jnp.sum(_log_sigmoid(osc_ref[...]))
                + jnp.sum(_log_sigmoid(-nsc_ref[...])))
        out_ref[...] = scale * jnp.full((1, 1), part, jnp.float32)

    return pl.pallas_call(
        body,
        in_specs=[
            pl.BlockSpec((ro, 128), lambda: (0, 0)),
            pl.BlockSpec((rn, 128), lambda: (0, 0)),
        ],
        out_specs=pl.BlockSpec((1, 1), lambda: (0, 0)),
        out_shape=jax.ShapeDtypeStruct((1, 1), jnp.float32),
    )


def kernel(iword, owords, nwords, emb_i, emb_o):
    V, D = emb_i.shape
    B, C = owords.shape
    NTOT = nwords.shape[1]
    iwf = iword.astype(jnp.int32)
    owf = owords.reshape(-1).astype(jnp.int32)
    nwf = nwords.reshape(-1).astype(jnp.int32)
    iv_all = jnp.take(emb_i, iwf, axis=0)
    osc, nsc = _make_sc_scores(V, D, B, C, NTOT)(
        iv_all, emb_o, owf, nwf)
    out = _make_loss(B, C, NTOT)(
        osc.reshape((B * C) // 128, 128), nsc.reshape((B * NTOT) // 128, 128))
    return out.reshape(())


# speedup vs baseline: 1.0099x; 1.0099x over previous
import functools

import jax
import jax.numpy as jnp
from jax import lax
from jax.experimental import pallas as pl
from jax.experimental.pallas import tpu as pltpu
from jax.experimental.pallas import tpu_sc as plsc

_NC = 2
_NS = 16
_NW = _NC * _NS
_L = 16


@functools.lru_cache(maxsize=None)
def _make_sc_scores(V, D, B, C, NTOT):
    ni = B // _NW
    no = (B * C) // _NW
    nn = (B * NTOT) // _NW
    NH = D // (2 * _L)
    assert nn == ni * NTOT and no == ni * C
    mesh = plsc.VectorSubcoreMesh(core_axis_name="c", subcore_axis_name="s")

    @functools.partial(
        pl.kernel, mesh=mesh,
        out_type=[
            jax.ShapeDtypeStruct((B * C,), jnp.float32),
            jax.ShapeDtypeStruct((B * NTOT,), jnp.float32),
        ],
        scratch_types=[
            pltpu.VMEM((no,), jnp.int32),
            pltpu.VMEM((nn,), jnp.int32),
            pltpu.VMEM((ni, D), jnp.float32),
            pltpu.VMEM((no, D), jnp.float32),
            pltpu.VMEM((NTOT, D), jnp.float32),
            pltpu.VMEM((NTOT, D), jnp.float32),
            pltpu.VMEM((no + _L,), jnp.float32),
            pltpu.VMEM((nn + _L,), jnp.float32),
            pltpu.SemaphoreType.DMA,
            pltpu.SemaphoreType.DMA,
            pltpu.SemaphoreType.DMA,
            pltpu.SemaphoreType.DMA,
            pltpu.SemaphoreType.DMA,
        ],
        compiler_params=pltpu.CompilerParams(
            use_tc_tiling_on_sc=False, needs_layout_passes=False),
    )
    def sgns_sc(iv_all, emb_o, ow, nw, osc_out, nsc_out,
                owi, nwi, iv_v, ow_v, nv0, nv1, sc_o, sc_n,
                semp, sem0, sem0b, sem1, sem1b):
        HK = NTOT // 2

        def start_nv(b, nv_buf, sems):
            base = b * NTOT
            pltpu.async_copy(
                emb_o.at[nwi.at[pl.ds(base, HK)]],
                nv_buf.at[pl.ds(0, HK)], sems[0])
            pltpu.async_copy(
                emb_o.at[nwi.at[pl.ds(base + HK, HK)]],
                nv_buf.at[pl.ds(HK, HK)], sems[1])

        def wait_nv(nv_buf, sems):
            pltpu.make_async_copy(
                emb_o.at[nwi.at[pl.ds(0, HK)]],
                nv_buf.at[pl.ds(0, HK)], sems[0]).wait()
            pltpu.make_async_copy(
                emb_o.at[nwi.at[pl.ds(0, HK)]],
                nv_buf.at[pl.ds(HK, HK)], sems[1]).wait()

        wid = lax.axis_index("s") * _NC + lax.axis_index("c")
        pltpu.sync_copy(nw.at[pl.ds(wid * nn, nn)], nwi)
        start_nv(0, nv0, (sem0, sem0b))
        pltpu.sync_copy(ow.at[pl.ds(wid * no, no)], owi)
        pltpu.sync_copy(iv_all.at[pl.ds(wid * ni, ni)], iv_v)
        pltpu.async_copy(emb_o.at[owi], ow_v, semp).wait()

        lane = lax.broadcasted_iota(jnp.int32, (_L,), 0)

        def row_quarters(rows_ref, r):
            return [rows_ref[r, pl.ds(q * _L, _L)] for q in range(2 * NH)]

        def dots_group(rows_ref, rbase, count, ivq, sc_ref, sbase):
            score = jnp.zeros((_L,), jnp.float32)
            for u in range(count):
                qs = row_quarters(rows_ref, rbase + u)
                p = qs[0] * ivq[0]
                for q in range(1, 2 * NH):
                    p += qs[q] * ivq[q]
                score = jnp.where(lane == u, jnp.sum(p), score)
            sc_ref[pl.ds(sbase, _L)] = score

        def half(b, nv_cur, sem_cur, nv_nxt, sem_nxt, nxt_b, has_next):
            @pl.when(has_next)
            def _():
                start_nv(nxt_b, nv_nxt, sem_nxt)
            ivq = row_quarters(iv_v, b)
            for g0 in range(0, C, _L):
                dots_group(ow_v, b * C + g0, min(_L, C - g0), ivq,
                           sc_o, b * C + g0)
            wait_nv(nv_cur, sem_cur)

            def gbody(jj, cc):
                dots_group(nv_cur, jj * _L, _L, ivq, sc_n,
                           b * NTOT + jj * _L)
                return cc
            lax.fori_loop(0, NTOT // _L, gbody, 0)

        def pair(bb, c):
            b0 = 2 * bb
            half(b0, nv0, (sem0, sem0b), nv1, (sem1, sem1b), b0 + 1, True)
            half(b0 + 1, nv1, (sem1, sem1b), nv0, (sem0, sem0b), b0 + 2,
                 bb < ni // 2 - 1)
            return c

        lax.fori_loop(0, ni // 2, pair, 0)
        pltpu.sync_copy(sc_o.at[pl.ds(0, no)],
                        osc_out.at[pl.ds(wid * no, no)])
        pltpu.sync_copy(sc_n.at[pl.ds(0, nn)],
                        nsc_out.at[pl.ds(wid * nn, nn)])

    return sgns_sc


def _log_sigmoid(x):
    return jnp.minimum(x, 0.0) - jnp.log1p(jnp.exp(-jnp.abs(x)))


@functools.lru_cache(maxsize=None)
def _make_loss(B, C, NTOT):
    scale = -1.0 / (B * C)
    ro = (B * C) // 128
    rn = (B * NTOT) // 128

    def body(osc_ref, nsc_ref, out_ref):
        part = (jnp.sum(_log_sigmoid(osc_ref[...]))
                + jnp.sum(_log_sigmoid(-nsc_ref[...])))
        out_ref[...] = scale * jnp.full((1, 1), part, jnp.float32)

    return pl.pallas_call(
        body,
        in_specs=[
            pl.BlockSpec((ro, 128), lambda: (0, 0)),
            pl.BlockSpec((rn, 128), lambda: (0, 0)),
        ],
        out_specs=pl.BlockSpec((1, 1), lambda: (0, 0)),
        out_shape=jax.ShapeDtypeStruct((1, 1), jnp.float32),
    )


def kernel(iword, owords, nwords, emb_i, emb_o):
    V, D = emb_i.shape
    B, C = owords.shape
    NTOT = nwords.shape[1]
    iwf = iword.astype(jnp.int32)
    owf = owords.reshape(-1).astype(jnp.int32)
    nwf = nwords.reshape(-1).astype(jnp.int32)
    iv_all = jnp.take(emb_i, iwf, axis=0)
    osc, nsc = _make_sc_scores(V, D, B, C, NTOT)(
        iv_all, emb_o, owf, nwf)
    out = _make_loss(B, C, NTOT)(
        osc.reshape((B * C) // 128, 128), nsc.reshape((B * NTOT) // 128, 128))
    return out.reshape(())
